# baseline (speedup 1.0000x reference)
import jax
import jax.numpy as jnp
from jax import lax
from jax.experimental import pallas as pl
from jax.experimental.pallas import tpu as pltpu
from jax.experimental.pallas import tpu_sc as plsc

F = 26
VOCAB = 100001
D = 32
B = 16384

_info = plsc.get_sparse_core_info()
NC = _info.num_cores
NS = _info.num_subcores
NW = NC * NS
H0 = 50048
H1 = VOCAB - H0
IDXC = 4096
UNROLL = 8


def _body(tab, xt, out, pln_a, pln_b, idx_a, idx_b, acc_v,
          sem_a, sem_b, sem_i0, sem_i1):
    d = lax.axis_index("s") * NC + lax.axis_index("c")

    zv = jnp.zeros((16,), jnp.float32)

    @plsc.parallel_loop(0, B, step=16, unroll=UNROLL)
    def _zero(i):
        acc_v[pl.ds(i, 16)] = zv

    def start_half(f, half):
        if half == 0:
            pltpu.async_copy(tab.at[f, d, pl.ds(0, H0)], pln_a, sem_a)
        else:
            pltpu.async_copy(tab.at[f, d, pl.ds(H0, H1)], pln_b, sem_b)

    def wait_half(half):
        if half == 0:
            pltpu.make_async_copy(tab.at[0, 0, pl.ds(0, H0)], pln_a, sem_a).wait()
        else:
            pltpu.make_async_copy(tab.at[0, 0, pl.ds(H0, H1)], pln_b, sem_b).wait()

    idx_bufs = (idx_a, idx_b)
    idx_sems = (sem_i0, sem_i1)
    IDXR = IDXC // 2048

    def start_idx(f, c):
        pltpu.async_copy(xt.at[pl.ds(f * 8 + c * IDXR, IDXR)],
                         idx_bufs[c % 2], idx_sems[c % 2])

    def wait_idx(c):
        pltpu.make_async_copy(xt.at[pl.ds(0, IDXR)],
                              idx_bufs[c % 2], idx_sems[c % 2]).wait()

    def gather_chunk(f, half, c):
        buf = pln_a if half == 0 else pln_b
        cur = idx_bufs[c % 2]
        for r in range(IDXR):
            base = c * IDXC + r * 2048

            @plsc.parallel_loop(0, 2048, step=16, unroll=UNROLL)
            def _gather(i, _buf=buf, _cur=cur, _base=base, _r=r, _half=half):
                v = _cur[_r, pl.ds(i, 16)]
                if _half == 0:
                    m = v < H0
                    vals = plsc.load_gather(_buf, [v], mask=m)
                else:
                    m = v >= H0
                    vals = plsc.load_gather(_buf, [v - H0], mask=m)
                vals = jnp.where(m, vals, 0.0)
                plsc.addupdate(acc_v.at[pl.ds(_base + i, 16)], vals)

    NCH = B // IDXC

    start_half(0, 0)
    start_half(0, 1)
    start_idx(0, 0)

    def fstep(f, carry):
        wait_half(0)
        for c in range(NCH):
            if c + 1 < NCH:
                start_idx(f, c + 1)
            wait_idx(c)
            gather_chunk(f, 0, c)

        wait_half(1)
        gather_chunk(f, 1, NCH - 2)
        start_idx(f, 0)
        gather_chunk(f, 1, NCH - 1)
        start_idx(f, 1)

        @pl.when(f + 1 < F)
        def _():
            start_half(f + 1, 0)

        for c in range(NCH - 2):
            wait_idx(c)
            gather_chunk(f, 1, c)
            if c + 2 < NCH - 2:
                start_idx(f, c + 2)

        @pl.when(f + 1 < F)
        def _():
            start_idx(f + 1, 0)

        @pl.when(f + 1 < F)
        def _():
            start_half(f + 1, 1)

        return carry

    lax.fori_loop(0, F, fstep, 0)
    pltpu.sync_copy(acc_v, out.at[d])


@jax.jit
def _run(tab_t, x_t):
    mesh = plsc.VectorSubcoreMesh(core_axis_name="c", subcore_axis_name="s")
    kfn = pl.kernel(
        _body,
        mesh=mesh,
        out_type=jax.ShapeDtypeStruct((D, B), jnp.float32),
        scratch_types=[
            pltpu.VMEM((H0,), jnp.float32),
            pltpu.VMEM((H1,), jnp.float32),
            pltpu.VMEM((IDXC // 2048, 2048), jnp.int32),
            pltpu.VMEM((IDXC // 2048, 2048), jnp.int32),
            pltpu.VMEM((B,), jnp.float32),
            pltpu.SemaphoreType.DMA,
            pltpu.SemaphoreType.DMA,
            pltpu.SemaphoreType.DMA,
            pltpu.SemaphoreType.DMA,
        ],
        compiler_params=pltpu.CompilerParams(needs_layout_passes=False),
    )
    return kfn(tab_t, x_t)


def kernel(X, tables):
    x_t = X.astype(jnp.int32).T.reshape(F * 8, 2048)
    tab_t = jnp.transpose(tables, (0, 2, 1))
    return _run(tab_t, x_t).T


# speedup vs baseline: 1.0783x; 1.0783x over previous
import jax
import jax.numpy as jnp
from jax import lax
from jax.experimental import pallas as pl
from jax.experimental.pallas import tpu as pltpu
from jax.experimental.pallas import tpu_sc as plsc

F = 26
VOCAB = 100001
D = 32
B = 16384

_info = plsc.get_sparse_core_info()
NC = _info.num_cores
NS = _info.num_subcores
NW = NC * NS
H0 = 50048
H1 = VOCAB - H0
IDXC = 4096
UNROLL = 8


def _body(tab, xt, out, pln_a, pln_b, idx_a, idx_b, acc_v,
          sem_a, sem_b, sem_i0, sem_i1):
    d = lax.axis_index("s") * NC + lax.axis_index("c")

    zv = jnp.zeros((16,), jnp.float32)

    @plsc.parallel_loop(0, B, step=16, unroll=UNROLL)
    def _zero(i):
        acc_v[pl.ds(i, 16)] = zv

    def start_half(f, half):
        if half == 0:
            pltpu.async_copy(tab.at[f, d, pl.ds(0, H0)], pln_a, sem_a)
        else:
            pltpu.async_copy(tab.at[f, d, pl.ds(H0, H1)], pln_b, sem_b)

    def wait_half(half):
        if half == 0:
            pltpu.make_async_copy(tab.at[0, 0, pl.ds(0, H0)], pln_a, sem_a).wait()
        else:
            pltpu.make_async_copy(tab.at[0, 0, pl.ds(H0, H1)], pln_b, sem_b).wait()

    idx_bufs = (idx_a, idx_b)
    idx_sems = (sem_i0, sem_i1)
    IDXR = IDXC // 2048

    def start_idx(f, c):
        pltpu.async_copy(xt.at[pl.ds(f * 8 + c * IDXR, IDXR)],
                         idx_bufs[c % 2], idx_sems[c % 2])

    def wait_idx(c):
        pltpu.make_async_copy(xt.at[pl.ds(0, IDXR)],
                              idx_bufs[c % 2], idx_sems[c % 2]).wait()

    def gather_chunk(f, half, c):
        buf = pln_a if half == 0 else pln_b
        cur = idx_bufs[c % 2]
        for r in range(IDXR):
            base = c * IDXC + r * 2048

            @plsc.parallel_loop(0, 2048, step=16, unroll=UNROLL)
            def _gather(i, _buf=buf, _cur=cur, _base=base, _r=r, _half=half):
                v = _cur[_r, pl.ds(i, 16)]
                if _half == 0:
                    m = v < H0
                    vals = plsc.load_gather(_buf, [v], mask=m)
                else:
                    m = v >= H0
                    vals = plsc.load_gather(_buf, [v - H0], mask=m)
                vals = jnp.where(m, vals, 0.0)
                plsc.addupdate(acc_v.at[pl.ds(_base + i, 16)], vals)

    NCH = B // IDXC

    start_half(0, 0)
    start_half(0, 1)
    start_idx(0, 0)

    def fstep(f, carry):
        wait_half(0)
        for c in range(NCH):
            if c + 1 < NCH:
                start_idx(f, c + 1)
            wait_idx(c)
            gather_chunk(f, 0, c)

        @pl.when(f + 1 < F)
        def _():
            start_half(f + 1, 0)

        wait_half(1)
        gather_chunk(f, 1, NCH - 2)
        start_idx(f, 0)
        gather_chunk(f, 1, NCH - 1)
        start_idx(f, 1)
        for c in range(NCH - 2):
            wait_idx(c)
            gather_chunk(f, 1, c)
            if c + 2 < NCH - 2:
                start_idx(f, c + 2)

        @pl.when(f + 1 < F)
        def _():
            start_half(f + 1, 1)

        @pl.when(f + 1 < F)
        def _():
            start_idx(f + 1, 0)

        return carry

    lax.fori_loop(0, F, fstep, 0)
    pltpu.sync_copy(acc_v, out.at[d])


@jax.jit
def _run(tab_t, x_t):
    mesh = plsc.VectorSubcoreMesh(core_axis_name="c", subcore_axis_name="s")
    kfn = pl.kernel(
        _body,
        mesh=mesh,
        out_type=jax.ShapeDtypeStruct((D, B), jnp.float32),
        scratch_types=[
            pltpu.VMEM((H0,), jnp.float32),
            pltpu.VMEM((H1,), jnp.float32),
            pltpu.VMEM((IDXC // 2048, 2048), jnp.int32),
            pltpu.VMEM((IDXC // 2048, 2048), jnp.int32),
            pltpu.VMEM((B,), jnp.float32),
            pltpu.SemaphoreType.DMA,
            pltpu.SemaphoreType.DMA,
            pltpu.SemaphoreType.DMA,
            pltpu.SemaphoreType.DMA,
        ],
        compiler_params=pltpu.CompilerParams(needs_layout_passes=False),
    )
    return kfn(tab_t, x_t)


def kernel(X, tables):
    x_t = X.astype(jnp.int32).T.reshape(F * 8, 2048)
    tab_t = jnp.transpose(tables, (0, 2, 1))
    return _run(tab_t, x_t).T
